# baseline (speedup 1.0000x reference)
import functools

import jax
import jax.numpy as jnp
from jax import lax
from jax.experimental import pallas as pl
from jax.experimental.pallas import tpu as pltpu
from jax.experimental.pallas import tpu_sc as plsc

_LANES = 16
_NBUF = 4
_CHUNK = 8
_SC_FRAC = 16


def _tc_rotate(xf, A, Bm, svec, cvec, t_hi):
    T, D = xf.shape
    P2 = A.shape[1]
    BLK = 1024

    def body(x_ref, a_ref, b_ref, s_ref, c_ref, o_ref):
        xb = x_ref[...]
        z = jnp.dot(xb.astype(jnp.bfloat16), a_ref[...].astype(jnp.bfloat16),
                    preferred_element_type=jnp.float32)
        z = z * s_ref[...]
        scat = jnp.dot(z.astype(jnp.bfloat16),
                       b_ref[...].astype(jnp.bfloat16),
                       preferred_element_type=jnp.float32)
        o_ref[...] = xb * c_ref[...] + scat

    return pl.pallas_call(
        body,
        grid=(t_hi // BLK,),
        in_specs=[
            pl.BlockSpec((BLK, D), lambda i: (i, 0)),
            pl.BlockSpec((D, P2), lambda i: (0, 0)),
            pl.BlockSpec((P2, D), lambda i: (0, 0)),
            pl.BlockSpec((1, P2), lambda i: (0, 0)),
            pl.BlockSpec((1, D), lambda i: (0, 0)),
        ],
        out_specs=pl.BlockSpec((BLK, D), lambda i: (i, 0)),
        out_shape=jax.ShapeDtypeStruct((t_hi, D), jnp.float32),
    )(xf, A, Bm, svec, cvec)


def _sc_rotate_tail(xf, pi, pj, cos, sin, t_lo):
    T, D = xf.shape
    xflat_arr = xf.reshape(T * D)
    NP = pi.shape[0]
    rows = T - t_lo
    NW = 32
    rows_per_w = rows // NW
    chunks = rows_per_w // _CHUNK
    CD = _CHUNK * D
    mesh = plsc.VectorSubcoreMesh(core_axis_name="c", subcore_axis_name="s")

    @functools.partial(
        pl.kernel,
        out_type=jax.ShapeDtypeStruct((rows * D,), jnp.float32),
        mesh=mesh,
        compiler_params=pltpu.CompilerParams(needs_layout_passes=False),
        scratch_types=(
            [pltpu.VMEM((CD,), jnp.float32) for _ in range(_NBUF)]
            + [pltpu.VMEM((NP,), jnp.int32),
               pltpu.VMEM((NP,), jnp.int32),
               pltpu.VMEM((NP,), jnp.float32),
               pltpu.VMEM((NP,), jnp.float32)]
            + [pltpu.SemaphoreType.DMA for _ in range(2 * _NBUF)]
        ),
    )
    def rot(x_hbm, pi_hbm, pj_hbm, cos_hbm, sin_hbm, out_hbm,
            b0, b1, b2, b3, piv, pjv, cosv, sinv,
            is0, is1, is2, is3, os0, os1, os2, os3):
        bufs = (b0, b1, b2, b3)
        isems = (is0, is1, is2, is3)
        osems = (os0, os1, os2, os3)
        wid = lax.axis_index("s") * 2 + lax.axis_index("c")
        pltpu.sync_copy(pi_hbm, piv)
        pltpu.sync_copy(pj_hbm, pjv)
        pltpu.sync_copy(cos_hbm, cosv)
        pltpu.sync_copy(sin_hbm, sinv)
        in_base = t_lo * D + wid * (rows_per_w * D)
        out_base = wid * (rows_per_w * D)

        iis = [piv[pl.ds(_LANES * v, _LANES)] for v in range(NP // _LANES)]
        jjs = [pjv[pl.ds(_LANES * v, _LANES)] for v in range(NP // _LANES)]
        ccs = [cosv[pl.ds(_LANES * v, _LANES)] for v in range(NP // _LANES)]
        sss = [sinv[pl.ds(_LANES * v, _LANES)] for v in range(NP // _LANES)]

        def start_in(gg, b):
            pltpu.async_copy(x_hbm.at[pl.ds(in_base + gg * CD, CD)],
                             bufs[b], isems[b])

        def wait_in(b):
            pltpu.make_async_copy(x_hbm.at[pl.ds(in_base, CD)],
                                  bufs[b], isems[b]).wait()

        def start_out(gg, b):
            pltpu.async_copy(bufs[b],
                             out_hbm.at[pl.ds(out_base + gg * CD, CD)],
                             osems[b])

        def wait_out(b):
            pltpu.make_async_copy(bufs[b],
                                  out_hbm.at[pl.ds(out_base, CD)],
                                  osems[b]).wait()

        start_in(0, 0)
        start_in(1, 1)

        @pl.loop(0, chunks, step=_NBUF)
        def _ring(g):
            for b in range(_NBUF):
                gg = g + b
                wait_in(b)
                for v in range(NP // _LANES):
                    ii, jj, cc, ss = iis[v], jjs[v], ccs[v], sss[v]
                    for r in range(_CHUNK):
                        io = ii + r * D
                        jo = jj + r * D
                        xi = plsc.load_gather(bufs[b], [io])
                        xj = plsc.load_gather(bufs[b], [jo])
                        plsc.store_scatter(bufs[b], [io], xi * cc - xj * ss)
                        plsc.store_scatter(bufs[b], [jo], xi * ss + xj * cc)
                start_out(gg, b)
                b2 = (b + 2) % _NBUF

                @pl.when(gg + 2 < chunks)
                def _():
                    @pl.when(gg >= 2)
                    def _():
                        wait_out(b2)
                    start_in(gg + 2, b2)

        for b in range(_NBUF):
            wait_out(b)

    return rot(xflat_arr, pi, pj, cos, sin).reshape(rows, D)


def kernel(x, angles, plane_i, plane_j):
    B, S, D = x.shape
    T = B * S
    NP = angles.shape[0]
    P2 = 2 * NP

    cos = jnp.cos(angles).astype(jnp.float32)
    sin = jnp.sin(angles).astype(jnp.float32)
    pi = plane_i.astype(jnp.int32)
    pj = plane_j.astype(jnp.int32)

    src = jnp.concatenate([pj, pi])
    dst = jnp.concatenate([pi, pj])
    A = jax.nn.one_hot(src, D, dtype=jnp.float32).T
    Bm = jax.nn.one_hot(dst, D, dtype=jnp.float32)
    svec = jnp.concatenate([-sin, sin]).reshape(1, P2)
    cvec = jnp.ones((D,), jnp.float32).at[pi].set(cos).at[pj].set(cos)
    cvec = cvec.reshape(1, D)

    xf = x.reshape(T, D)
    t_split = T - T // _SC_FRAC

    tc_out = _tc_rotate(xf, A, Bm, svec, cvec, t_split)
    sc_out = _sc_rotate_tail(xf, pi, pj, cos, sin, t_split)
    out = jnp.concatenate([tc_out, sc_out], axis=0)
    return out.reshape(B, S, D)


# speedup vs baseline: 3.0543x; 3.0543x over previous
import jax
import jax.numpy as jnp
from jax.experimental import pallas as pl


def kernel(x, angles, plane_i, plane_j):
    B, S, D = x.shape
    T = B * S
    NP = angles.shape[0]
    P2 = 2 * NP

    cos = jnp.cos(angles).astype(jnp.float32)
    sin = jnp.sin(angles).astype(jnp.float32)
    pi = plane_i.astype(jnp.int32)
    pj = plane_j.astype(jnp.int32)

    src = jnp.concatenate([pj, pi])
    dst = jnp.concatenate([pi, pj])
    A = jax.nn.one_hot(src, D, dtype=jnp.float32).T
    Bm = jax.nn.one_hot(dst, D, dtype=jnp.float32)
    svec = jnp.concatenate([-sin, sin]).reshape(1, P2)
    cvec = jnp.ones((D,), jnp.float32).at[pi].set(cos).at[pj].set(cos)
    cvec = cvec.reshape(1, D)

    xf = x.reshape(T, D)
    BLK = 1024
    grid = (T // BLK,)

    def body(x_ref, a_ref, b_ref, s_ref, c_ref, o_ref):
        xb = x_ref[...]
        z = jnp.dot(xb.astype(jnp.bfloat16), a_ref[...].astype(jnp.bfloat16),
                    preferred_element_type=jnp.float32)
        z = z * s_ref[...]
        scat = jnp.dot(z.astype(jnp.bfloat16),
                       b_ref[...].astype(jnp.bfloat16),
                       preferred_element_type=jnp.float32)
        o_ref[...] = xb * c_ref[...] + scat

    out = pl.pallas_call(
        body,
        grid=grid,
        in_specs=[
            pl.BlockSpec((BLK, D), lambda i: (i, 0)),
            pl.BlockSpec((D, P2), lambda i: (0, 0)),
            pl.BlockSpec((P2, D), lambda i: (0, 0)),
            pl.BlockSpec((1, P2), lambda i: (0, 0)),
            pl.BlockSpec((1, D), lambda i: (0, 0)),
        ],
        out_specs=pl.BlockSpec((BLK, D), lambda i: (i, 0)),
        out_shape=jax.ShapeDtypeStruct((T, D), jnp.float32),
    )(xf, A, Bm, svec, cvec)
    return out.reshape(B, S, D)


# speedup vs baseline: 3.2273x; 1.0567x over previous
import jax
import jax.numpy as jnp
from jax.experimental import pallas as pl


def kernel(x, angles, plane_i, plane_j):
    B, S, D = x.shape
    T = B * S
    NP = angles.shape[0]
    P2 = 2 * NP

    cos = jnp.cos(angles).astype(jnp.float32)
    sin = jnp.sin(angles).astype(jnp.float32)
    pi = plane_i.astype(jnp.int32)
    pj = plane_j.astype(jnp.int32)

    selfc = jnp.concatenate([pi, pj])
    partc = jnp.concatenate([pj, pi])
    As = jax.nn.one_hot(selfc, D, dtype=jnp.float32).T
    Ap = jax.nn.one_hot(partc, D, dtype=jnp.float32).T
    Bm = jax.nn.one_hot(selfc, D, dtype=jnp.float32)
    ws = jnp.concatenate([cos - 1.0, cos - 1.0]).reshape(1, P2)
    wp = jnp.concatenate([-sin, sin]).reshape(1, P2)

    xf = x.reshape(T, D)
    BLK = 1024
    grid = (T // BLK,)

    def body(x_ref, as_ref, ap_ref, b_ref, ws_ref, wp_ref, o_ref):
        xb = x_ref[...]
        xb16 = xb.astype(jnp.bfloat16)
        zs = jnp.dot(xb16, as_ref[...].astype(jnp.bfloat16),
                     preferred_element_type=jnp.float32)
        zp = jnp.dot(xb16, ap_ref[...].astype(jnp.bfloat16),
                     preferred_element_type=jnp.float32)
        dv = zs * ws_ref[...] + zp * wp_ref[...]
        delta = jnp.dot(dv.astype(jnp.bfloat16),
                        b_ref[...].astype(jnp.bfloat16),
                        preferred_element_type=jnp.float32)
        o_ref[...] = xb + delta

    out = pl.pallas_call(
        body,
        grid=grid,
        in_specs=[
            pl.BlockSpec((BLK, D), lambda i: (i, 0)),
            pl.BlockSpec((D, P2), lambda i: (0, 0)),
            pl.BlockSpec((D, P2), lambda i: (0, 0)),
            pl.BlockSpec((P2, D), lambda i: (0, 0)),
            pl.BlockSpec((1, P2), lambda i: (0, 0)),
            pl.BlockSpec((1, P2), lambda i: (0, 0)),
        ],
        out_specs=pl.BlockSpec((BLK, D), lambda i: (i, 0)),
        out_shape=jax.ShapeDtypeStruct((T, D), jnp.float32),
    )(xf, As, Ap, Bm, ws, wp)
    return out.reshape(B, S, D)
